# baseline (speedup 1.0000x reference)
import functools

import jax
import jax.numpy as jnp
from jax import lax
from jax.experimental import pallas as pl
from jax.experimental.pallas import tpu as pltpu
from jax.experimental.pallas import tpu_sc as plsc

_VOCAB = 1000000
_EMBED = 64
_BATCH = 4096
_SEQ = 200

_NC = 2
_NS = 16
_NW = _NC * _NS
_CHUNK = 128
_ROWS = _BATCH * _SEQ
_PER_W = _ROWS // _NW
_GPW = _PER_W // _CHUNK


def _pos_table(seq_len, hidden, max_wavelength=10000.0):
    position = jnp.arange(seq_len, dtype=jnp.float32)
    min_freq = 1.0 / max_wavelength
    timescales = jnp.power(
        min_freq,
        (2.0 * (jnp.arange(hidden) // 2).astype(jnp.float32)) / float(hidden),
    )
    angles = position[:, None] * timescales[None, :]
    cos_mask = (jnp.arange(hidden) % 2).astype(jnp.float32)
    sin_mask = 1.0 - cos_mask
    return jnp.sin(angles) * sin_mask + jnp.cos(angles) * cos_mask


_mesh = plsc.VectorSubcoreMesh(core_axis_name="c", subcore_axis_name="s")


@functools.partial(
    pl.kernel,
    mesh=_mesh,
    compiler_params=pltpu.CompilerParams(use_tc_tiling_on_sc=False),
    out_type=jax.ShapeDtypeStruct((_ROWS, _EMBED), jnp.float32),
    scratch_types=[
        pltpu.VMEM((_GPW, _CHUNK), jnp.int32),
        pltpu.VMEM((_SEQ, _EMBED), jnp.float32),
        pltpu.VMEM((_CHUNK, _EMBED), jnp.float32),
        pltpu.SemaphoreType.DMA,
    ],
)
def _sc_embed(table_hbm, idx_hbm, pos_hbm, out_hbm, idx_v, pos_v, rows_v, sem):
    wid = lax.axis_index("s") * _NC + lax.axis_index("c")
    base = wid * _PER_W
    pltpu.sync_copy(idx_hbm.at[wid], idx_v)
    pltpu.sync_copy(pos_hbm, pos_v)

    def chunk_body(g, _):
        pltpu.async_copy(table_hbm.at[idx_v.at[g]], rows_v, sem).wait()

        def row_body(i, _):
            l = lax.rem(g * _CHUNK + i, _SEQ)
            for j in range(_EMBED // 16):
                s = pl.ds(j * 16, 16)
                rows_v[i, s] = rows_v[i, s] + pos_v[l, s]
            return 0

        lax.fori_loop(0, _CHUNK, row_body, 0)
        pltpu.sync_copy(rows_v, out_hbm.at[pl.ds(base + g * _CHUNK, _CHUNK)])
        return 0

    lax.fori_loop(0, _GPW, chunk_body, 0)


def kernel(x, table):
    idx = x.astype(jnp.int32).reshape(_NW, _GPW, _CHUNK)
    pos = _pos_table(_SEQ, _EMBED)
    out = _sc_embed(table, idx, pos)
    return out.reshape(_BATCH, _SEQ, _EMBED)


# speedup vs baseline: 1.3215x; 1.3215x over previous
import functools

import jax
import jax.numpy as jnp
from jax import lax
from jax.experimental import pallas as pl
from jax.experimental.pallas import tpu as pltpu
from jax.experimental.pallas import tpu_sc as plsc

_VOCAB = 1000000
_EMBED = 64
_BATCH = 4096
_SEQ = 200

_NC = 2
_NS = 16
_NW = _NC * _NS
_ROWS = _BATCH * _SEQ
_PER_W = _ROWS // _NW
_GPW = _PER_W // _SEQ


def _pos_table(seq_len, hidden, max_wavelength=10000.0):
    position = jnp.arange(seq_len, dtype=jnp.float32)
    min_freq = 1.0 / max_wavelength
    timescales = jnp.power(
        min_freq,
        (2.0 * (jnp.arange(hidden) // 2).astype(jnp.float32)) / float(hidden),
    )
    angles = position[:, None] * timescales[None, :]
    cos_mask = (jnp.arange(hidden) % 2).astype(jnp.float32)
    sin_mask = 1.0 - cos_mask
    return jnp.sin(angles) * sin_mask + jnp.cos(angles) * cos_mask


_mesh = plsc.VectorSubcoreMesh(core_axis_name="c", subcore_axis_name="s")


@functools.partial(
    pl.kernel,
    mesh=_mesh,
    compiler_params=pltpu.CompilerParams(use_tc_tiling_on_sc=False),
    out_type=jax.ShapeDtypeStruct((_ROWS, _EMBED), jnp.float32),
    scratch_types=[
        pltpu.VMEM((_GPW, _SEQ), jnp.int32),
        pltpu.VMEM_SHARED((_SEQ, _EMBED), jnp.float32),
        pltpu.VMEM((_SEQ, _EMBED), jnp.float32),
        pltpu.SemaphoreType.DMA,
    ],
)
def _sc_embed(table_hbm, idx_hbm, pos_hbm, out_hbm, idx_v, pos_sh, rows_v, sem):
    sid = lax.axis_index("s")
    wid = sid * _NC + lax.axis_index("c")
    base = wid * _PER_W
    pltpu.sync_copy(idx_hbm.at[wid], idx_v)

    @pl.when(sid == 0)
    def _():
        pltpu.sync_copy(pos_hbm, pos_sh)

    plsc.subcore_barrier()

    def chunk_body(g, _):
        pltpu.sync_copy(pos_sh, rows_v)
        a = pltpu.async_copy(
            table_hbm.at[idx_v.at[g, pl.ds(0, 128)]],
            rows_v.at[pl.ds(0, 128)], sem, add=True)
        b = pltpu.async_copy(
            table_hbm.at[idx_v.at[g, pl.ds(128, _SEQ - 128)]],
            rows_v.at[pl.ds(128, _SEQ - 128)], sem, add=True)
        a.wait()
        b.wait()
        pltpu.sync_copy(rows_v, out_hbm.at[pl.ds(base + g * _SEQ, _SEQ)])
        return 0

    lax.fori_loop(0, _GPW, chunk_body, 0)


def kernel(x, table):
    idx = x.astype(jnp.int32).reshape(_NW, _GPW, _SEQ)
    pos = _pos_table(_SEQ, _EMBED)
    out = _sc_embed(table, idx, pos)
    return out.reshape(_BATCH, _SEQ, _EMBED)
